# baseline (speedup 1.0000x reference)
import functools

import jax
import jax.numpy as jnp
from jax import lax
from jax.experimental import pallas as pl
from jax.experimental.pallas import tpu as pltpu
from jax.experimental.pallas import tpu_sc as plsc

E = 16
D_MODEL = 768
D_FF = 3072
N_TOK = 4096
K_TOP = 2
NPAIR = N_TOK * K_TOP
TILE = 128
NP_PAD = NPAIR + E * TILE
NT = NP_PAD // TILE

NC = 2
NS = 16
LANES = 16


_RT = 512


def _router_body(x_ref, wr_ref, eo_ref, go_ref):
    logits = jnp.dot(x_ref[...], wr_ref[...].T, preferred_element_type=jnp.float32)
    probs = jax.nn.softmax(logits, axis=-1)
    n = probs.shape[0]
    ids = lax.broadcasted_iota(jnp.int32, (n, E), 1)
    m1 = jnp.max(probs, axis=-1, keepdims=True)
    a1 = jnp.min(jnp.where(probs == m1, ids, E), axis=-1, keepdims=True)
    masked = jnp.where(ids == a1, -jnp.inf, probs)
    m2 = jnp.max(masked, axis=-1, keepdims=True)
    a2 = jnp.min(jnp.where(masked == m2, ids, E), axis=-1, keepdims=True)
    eo_ref[...] = jnp.concatenate([a1.T, a2.T], axis=0)
    go_ref[...] = jnp.concatenate([m1.T, m2.T], axis=0)


@jax.jit
def _router(x2d, W_router):
    grid = (N_TOK // _RT,)
    return pl.pallas_call(
        _router_body,
        grid=grid,
        in_specs=[
            pl.BlockSpec((_RT, D_MODEL), lambda t: (t, 0)),
            pl.BlockSpec((E, D_MODEL), lambda t: (0, 0)),
        ],
        out_specs=[
            pl.BlockSpec((K_TOP, _RT), lambda t: (0, t)),
            pl.BlockSpec((K_TOP, _RT), lambda t: (0, t)),
        ],
        out_shape=[
            jax.ShapeDtypeStruct((K_TOP, N_TOK), jnp.int32),
            jax.ShapeDtypeStruct((K_TOP, N_TOK), jnp.float32),
        ],
    )(x2d, W_router)


_CHUNK = NPAIR // NS
_VR = _CHUNK // LANES
_ROWS = _CHUNK // NC
_RSUB = 64
_NRB = _ROWS // _RSUB


def _dispatch_kernel(epair_hbm, x_hbm, xs_hbm, pos_hbm, te_hbm, stage_hbm,
                     ids_v, histall_v, cnt_v, start_v, ebase_v,
                     pos_v, posb_v, tok_v, row_v, te_v, sem):
    cid = lax.axis_index("c")
    sid = lax.axis_index("s")
    base = sid * _CHUNK
    lane_iota = lax.iota(jnp.int32, LANES)
    zero16 = jnp.zeros((LANES,), jnp.int32)

    pltpu.sync_copy(epair_hbm.at[pl.ds(base, _CHUNK)], ids_v)

    d0, _ = plsc.scan_count(zero16)
    off = d0 - lane_iota

    cnt_v[...] = zero16

    def rank_body(j, carry):
        v = ids_v[pl.ds(j * LANES, LANES)]
        dup, last = plsc.scan_count(v)
        b = plsc.load_gather(cnt_v, [v])
        p = b + dup - off
        pos_v[pl.ds(j * LANES, LANES)] = p
        plsc.store_scatter(cnt_v, [v], p + 1, mask=last)
        tok_v[pl.ds(j * LANES, LANES)] = (base + j * LANES + lane_iota) & (N_TOK - 1)
        return carry

    lax.fori_loop(0, _VR, rank_body, 0)

    pltpu.sync_copy(cnt_v, stage_hbm.at[cid, sid])
    plsc.subcore_barrier()
    pltpu.sync_copy(stage_hbm.at[cid], histall_v)

    totals = zero16
    before = zero16
    for w in range(NS):
        hw = histall_v[w]
        totals = totals + hw
        before = before + jnp.where(w < sid, hw, zero16)

    padded = (totals + (TILE - 1)) & (-TILE)
    ebase = plsc.cumsum(padded) - padded
    ebase_v[...] = ebase
    start_v[...] = ebase + before

    def fin_body(j, carry):
        v = ids_v[pl.ds(j * LANES, LANES)]
        p = pos_v[pl.ds(j * LANES, LANES)]
        gp = plsc.load_gather(start_v, [v]) + p
        pos_v[pl.ds(j * LANES, LANES)] = gp
        return carry

    lax.fori_loop(0, _VR, fin_body, 0)

    for b in range(NC * _NRB):
        for jj in range(_RSUB // LANES):
            posb_v[b, pl.ds(jj * LANES, LANES)] = \
                pos_v[pl.ds(b * _RSUB + jj * LANES, LANES)]

    @pl.when(cid == 0)
    def _write_pos():
        pltpu.sync_copy(pos_v, pos_hbm.at[pl.ds(base, _CHUNK)])

    for rb in range(_NRB):
        b = cid * _NRB + rb
        pltpu.async_copy(
            x_hbm.at[tok_v.at[pl.ds(b * _RSUB, _RSUB)]], row_v, sem).wait()
        pltpu.async_copy(row_v, xs_hbm.at[posb_v.at[b]], sem).wait()

    @pl.when(jnp.logical_and(cid == 0, sid == 0))
    def _tiles():
        for j in range(NT // LANES):
            tstart = (lane_iota + j * LANES) * TILE
            acc = jnp.full((LANES,), -1, jnp.int32)
            for e in range(E):
                be = jnp.take_along_axis(
                    ebase, jnp.full((LANES,), e, jnp.int32), axis=0)
                acc = acc + jnp.where(tstart >= be, 1, 0)
            te_v[pl.ds(j * LANES, LANES)] = acc
        pltpu.sync_copy(te_v, te_hbm)


@jax.jit
def _dispatch(epair, x2d):
    mesh = plsc.VectorSubcoreMesh(core_axis_name="c", subcore_axis_name="s", num_cores=NC, num_subcores=NS)
    f = pl.kernel(
        _dispatch_kernel,
        mesh=mesh,
        compiler_params=pltpu.CompilerParams(needs_layout_passes=False),
        out_type=[
            jax.ShapeDtypeStruct((NP_PAD, D_MODEL), jnp.float32),
            jax.ShapeDtypeStruct((NPAIR,), jnp.int32),
            jax.ShapeDtypeStruct((NT,), jnp.int32),
            jax.ShapeDtypeStruct((NC, NS, LANES), jnp.int32),
        ],
        scratch_types=[
            pltpu.VMEM((_CHUNK,), jnp.int32),
            pltpu.VMEM((NS, LANES), jnp.int32),
            pltpu.VMEM((LANES,), jnp.int32),
            pltpu.VMEM((LANES,), jnp.int32),
            pltpu.VMEM((LANES,), jnp.int32),
            pltpu.VMEM((_CHUNK,), jnp.int32),
            pltpu.VMEM((_NRB * NC, _RSUB), jnp.int32),
            pltpu.VMEM((_CHUNK,), jnp.int32),
            pltpu.VMEM((_RSUB, D_MODEL), jnp.float32),
            pltpu.VMEM((NT,), jnp.int32),
            pltpu.SemaphoreType.DMA,
        ],
    )
    return f(epair, x2d)


def _gg_body(te_ref, xs_ref, w1_ref, b1_ref, w2_ref, b2_ref, ys_ref):
    dn = (((1,), (1,)), ((), ()))
    xb = xs_ref[...].astype(jnp.bfloat16)
    h = lax.dot_general(xb, w1_ref[0].astype(jnp.bfloat16), dn,
                        preferred_element_type=jnp.float32)
    h = h + b1_ref[0]
    h = 0.5 * h * (1.0 + lax.erf(h * 0.7071067811865476))
    o = lax.dot_general(h.astype(jnp.bfloat16), w2_ref[0].astype(jnp.bfloat16),
                        dn, preferred_element_type=jnp.float32)
    ys_ref[...] = o + b2_ref[0]


@jax.jit
def _grouped_gemm(te, xs, W1, b1, W2, b2):
    grid_spec = pltpu.PrefetchScalarGridSpec(
        num_scalar_prefetch=1,
        grid=(NT,),
        in_specs=[
            pl.BlockSpec((TILE, D_MODEL), lambda t, te: (t, 0)),
            pl.BlockSpec((1, D_FF, D_MODEL), lambda t, te: (te[t], 0, 0)),
            pl.BlockSpec((1, 1, D_FF), lambda t, te: (te[t], 0, 0)),
            pl.BlockSpec((1, D_MODEL, D_FF), lambda t, te: (te[t], 0, 0)),
            pl.BlockSpec((1, 1, D_MODEL), lambda t, te: (te[t], 0, 0)),
        ],
        out_specs=pl.BlockSpec((TILE, D_MODEL), lambda t, te: (t, 0)),
    )
    return pl.pallas_call(
        _gg_body,
        grid_spec=grid_spec,
        out_shape=jax.ShapeDtypeStruct((NP_PAD, D_MODEL), jnp.float32),
    )(te, xs, W1, b1.reshape(E, 1, D_FF), W2, b2.reshape(E, 1, D_MODEL))


_CTOK = N_TOK // (NC * NS)
_CSUB = 32
_NCB = _CTOK // _CSUB


def _combine_kernel(x_hbm, ys_hbm, pos_hbm, g_hbm, out_hbm,
                    p0_v, p1_v, g0_v, g1_v, bx_v, y0_v, y1_v, sem):
    cid = lax.axis_index("c")
    sid = lax.axis_index("s")
    wid = sid * NC + cid
    tbase = wid * _CTOK

    for sb in range(_NCB):
        t0 = tbase + sb * _CSUB
        pltpu.sync_copy(pos_hbm.at[pl.ds(t0, _CSUB)], p0_v.at[0])
        pltpu.sync_copy(pos_hbm.at[pl.ds(N_TOK + t0, _CSUB)], p1_v.at[0])
        pltpu.sync_copy(g_hbm.at[0, pl.ds(t0, _CSUB)], g0_v)
        pltpu.sync_copy(g_hbm.at[1, pl.ds(t0, _CSUB)], g1_v)
        pltpu.sync_copy(x_hbm.at[pl.ds(t0, _CSUB)], bx_v)
        pltpu.async_copy(ys_hbm.at[p0_v.at[0]], y0_v, sem).wait()
        pltpu.async_copy(ys_hbm.at[p1_v.at[0]], y1_v, sem).wait()

        def tok_body(t, carry):
            g0 = plsc.load_gather(g0_v, [jnp.full((LANES,), t, jnp.int32)])
            g1 = plsc.load_gather(g1_v, [jnp.full((LANES,), t, jnp.int32)])
            for v in range(D_MODEL // LANES):
                sl = pl.ds(v * LANES, LANES)
                bx_v[t, sl] = (bx_v[t, sl] + g0 * y0_v[t, sl]
                               + g1 * y1_v[t, sl])
            return carry

        lax.fori_loop(0, _CSUB, tok_body, 0)
        pltpu.sync_copy(bx_v, out_hbm.at[pl.ds(t0, _CSUB)])


@jax.jit
def _combine(x2d, ys, pos, g):
    mesh = plsc.VectorSubcoreMesh(core_axis_name="c", subcore_axis_name="s", num_cores=NC, num_subcores=NS)
    f = pl.kernel(
        _combine_kernel,
        mesh=mesh,
        compiler_params=pltpu.CompilerParams(needs_layout_passes=False),
        out_type=jax.ShapeDtypeStruct((N_TOK, D_MODEL), jnp.float32),
        scratch_types=[
            pltpu.VMEM((1, _CSUB), jnp.int32),
            pltpu.VMEM((1, _CSUB), jnp.int32),
            pltpu.VMEM((_CSUB,), jnp.float32),
            pltpu.VMEM((_CSUB,), jnp.float32),
            pltpu.VMEM((_CSUB, D_MODEL), jnp.float32),
            pltpu.VMEM((_CSUB, D_MODEL), jnp.float32),
            pltpu.VMEM((_CSUB, D_MODEL), jnp.float32),
            pltpu.SemaphoreType.DMA,
        ],
    )
    return f(x2d, ys, pos, g)


def kernel(x, W_router, W1, b1, W2, b2):
    B, L, D = x.shape
    x2d = x.reshape(B * L, D)
    e_top, g_top = _router(x2d, W_router)
    xs, pos, te, _ = _dispatch(e_top.reshape(NPAIR), x2d)
    ys = _grouped_gemm(te, xs, W1, b1, W2, b2)
    out = _combine(x2d, ys, pos, g_top)
    return out.reshape(B, L, D)


# speedup vs baseline: 1.3434x; 1.3434x over previous
import functools

import jax
import jax.numpy as jnp
from jax import lax
from jax.experimental import pallas as pl
from jax.experimental.pallas import tpu as pltpu
from jax.experimental.pallas import tpu_sc as plsc

E = 16
D_MODEL = 768
D_FF = 3072
N_TOK = 4096
K_TOP = 2
NPAIR = N_TOK * K_TOP
TILE = 256
NP_PAD = NPAIR + E * TILE
NT = NP_PAD // TILE

NC = 2
NS = 16
LANES = 16


_RT = 512


def _router_body(x_ref, wr_ref, eo_ref, go_ref):
    logits = jnp.dot(x_ref[...], wr_ref[...].T, preferred_element_type=jnp.float32)
    probs = jax.nn.softmax(logits, axis=-1)
    n = probs.shape[0]
    ids = lax.broadcasted_iota(jnp.int32, (n, E), 1)
    m1 = jnp.max(probs, axis=-1, keepdims=True)
    a1 = jnp.min(jnp.where(probs == m1, ids, E), axis=-1, keepdims=True)
    masked = jnp.where(ids == a1, -jnp.inf, probs)
    m2 = jnp.max(masked, axis=-1, keepdims=True)
    a2 = jnp.min(jnp.where(masked == m2, ids, E), axis=-1, keepdims=True)
    eo_ref[...] = jnp.concatenate([a1.T, a2.T], axis=0)
    go_ref[...] = jnp.concatenate([m1.T, m2.T], axis=0)


@jax.jit
def _router(x2d, W_router):
    grid = (N_TOK // _RT,)
    return pl.pallas_call(
        _router_body,
        grid=grid,
        in_specs=[
            pl.BlockSpec((_RT, D_MODEL), lambda t: (t, 0)),
            pl.BlockSpec((E, D_MODEL), lambda t: (0, 0)),
        ],
        out_specs=[
            pl.BlockSpec((K_TOP, _RT), lambda t: (0, t)),
            pl.BlockSpec((K_TOP, _RT), lambda t: (0, t)),
        ],
        out_shape=[
            jax.ShapeDtypeStruct((K_TOP, N_TOK), jnp.int32),
            jax.ShapeDtypeStruct((K_TOP, N_TOK), jnp.float32),
        ],
    )(x2d, W_router)


_CHUNK = NPAIR // NS
_VR = _CHUNK // LANES
_ROWS = _CHUNK // NC
_RSUB = 64
_NRB = _ROWS // _RSUB


def _dispatch_kernel(epair_hbm, x_hbm, xs_hbm, pos_hbm, te_hbm, stage_hbm,
                     ids_v, histall_v, cnt_v, start_v, ebase_v,
                     pos_v, posb_v, tok_v, row_v, te_v, sem):
    cid = lax.axis_index("c")
    sid = lax.axis_index("s")
    base = sid * _CHUNK
    lane_iota = lax.iota(jnp.int32, LANES)
    zero16 = jnp.zeros((LANES,), jnp.int32)

    pltpu.sync_copy(epair_hbm.at[pl.ds(base, _CHUNK)], ids_v)

    d0, _ = plsc.scan_count(zero16)
    off = d0 - lane_iota

    cnt_v[...] = zero16

    def rank_body(j, carry):
        v = ids_v[pl.ds(j * LANES, LANES)]
        dup, last = plsc.scan_count(v)
        b = plsc.load_gather(cnt_v, [v])
        p = b + dup - off
        pos_v[pl.ds(j * LANES, LANES)] = p
        plsc.store_scatter(cnt_v, [v], p + 1, mask=last)
        tok_v[pl.ds(j * LANES, LANES)] = (base + j * LANES + lane_iota) & (N_TOK - 1)
        return carry

    lax.fori_loop(0, _VR, rank_body, 0)

    pltpu.sync_copy(cnt_v, stage_hbm.at[cid, sid])
    plsc.subcore_barrier()
    pltpu.sync_copy(stage_hbm.at[cid], histall_v)

    totals = zero16
    before = zero16
    for w in range(NS):
        hw = histall_v[w]
        totals = totals + hw
        before = before + jnp.where(w < sid, hw, zero16)

    padded = (totals + (TILE - 1)) & (-TILE)
    ebase = plsc.cumsum(padded) - padded
    ebase_v[...] = ebase
    start_v[...] = ebase + before

    def fin_body(j, carry):
        v = ids_v[pl.ds(j * LANES, LANES)]
        p = pos_v[pl.ds(j * LANES, LANES)]
        gp = plsc.load_gather(start_v, [v]) + p
        pos_v[pl.ds(j * LANES, LANES)] = gp
        return carry

    lax.fori_loop(0, _VR, fin_body, 0)

    for b in range(NC * _NRB):
        for jj in range(_RSUB // LANES):
            posb_v[b, pl.ds(jj * LANES, LANES)] = \
                pos_v[pl.ds(b * _RSUB + jj * LANES, LANES)]

    @pl.when(cid == 0)
    def _write_pos():
        pltpu.sync_copy(pos_v, pos_hbm.at[pl.ds(base, _CHUNK)])

    for rb in range(_NRB):
        b = cid * _NRB + rb
        pltpu.async_copy(
            x_hbm.at[tok_v.at[pl.ds(b * _RSUB, _RSUB)]], row_v, sem).wait()
        pltpu.async_copy(row_v, xs_hbm.at[posb_v.at[b]], sem).wait()

    @pl.when(jnp.logical_and(cid == 0, sid == 0))
    def _tiles():
        for j in range(NT // LANES):
            tstart = (lane_iota + j * LANES) * TILE
            acc = jnp.full((LANES,), -1, jnp.int32)
            for e in range(E):
                be = jnp.take_along_axis(
                    ebase, jnp.full((LANES,), e, jnp.int32), axis=0)
                acc = acc + jnp.where(tstart >= be, 1, 0)
            te_v[pl.ds(j * LANES, LANES)] = acc
        pltpu.sync_copy(te_v, te_hbm)


@jax.jit
def _dispatch(epair, x2d):
    mesh = plsc.VectorSubcoreMesh(core_axis_name="c", subcore_axis_name="s", num_cores=NC, num_subcores=NS)
    f = pl.kernel(
        _dispatch_kernel,
        mesh=mesh,
        compiler_params=pltpu.CompilerParams(needs_layout_passes=False),
        out_type=[
            jax.ShapeDtypeStruct((NP_PAD, D_MODEL), jnp.float32),
            jax.ShapeDtypeStruct((NPAIR,), jnp.int32),
            jax.ShapeDtypeStruct((NT,), jnp.int32),
            jax.ShapeDtypeStruct((NC, NS, LANES), jnp.int32),
        ],
        scratch_types=[
            pltpu.VMEM((_CHUNK,), jnp.int32),
            pltpu.VMEM((NS, LANES), jnp.int32),
            pltpu.VMEM((LANES,), jnp.int32),
            pltpu.VMEM((LANES,), jnp.int32),
            pltpu.VMEM((LANES,), jnp.int32),
            pltpu.VMEM((_CHUNK,), jnp.int32),
            pltpu.VMEM((_NRB * NC, _RSUB), jnp.int32),
            pltpu.VMEM((_CHUNK,), jnp.int32),
            pltpu.VMEM((_RSUB, D_MODEL), jnp.float32),
            pltpu.VMEM((NT,), jnp.int32),
            pltpu.SemaphoreType.DMA,
        ],
    )
    return f(epair, x2d)


def _gg_body(te_ref, xs_ref, w1_ref, b1_ref, w2_ref, b2_ref, ys_ref,
             w1b_ref, w2b_ref, xb_ref, hb_ref):
    t = pl.program_id(0)
    dn = (((1,), (1,)), ((), ()))

    @pl.when(jnp.logical_or(
        t == 0, te_ref[t] != te_ref[jnp.maximum(t - 1, 0)]))
    def _recast():
        w1b_ref[...] = w1_ref[0].astype(jnp.bfloat16)
        w2b_ref[...] = w2_ref[0].astype(jnp.bfloat16)

    xb_ref[...] = xs_ref[...].astype(jnp.bfloat16)
    h = lax.dot_general(xb_ref[...], w1b_ref[...], dn,
                        preferred_element_type=jnp.float32)
    h = h + b1_ref[0]
    h = 0.5 * h * (1.0 + lax.erf(h * 0.7071067811865476))
    hb_ref[...] = h.astype(jnp.bfloat16)
    o = lax.dot_general(hb_ref[...], w2b_ref[...], dn,
                        preferred_element_type=jnp.float32)
    ys_ref[...] = o + b2_ref[0]


@jax.jit
def _grouped_gemm(te, xs, W1, b1, W2, b2):
    grid_spec = pltpu.PrefetchScalarGridSpec(
        num_scalar_prefetch=1,
        grid=(NT,),
        in_specs=[
            pl.BlockSpec((TILE, D_MODEL), lambda t, te: (t, 0)),
            pl.BlockSpec((1, D_FF, D_MODEL), lambda t, te: (te[t], 0, 0)),
            pl.BlockSpec((1, 1, D_FF), lambda t, te: (te[t], 0, 0)),
            pl.BlockSpec((1, D_MODEL, D_FF), lambda t, te: (te[t], 0, 0)),
            pl.BlockSpec((1, 1, D_MODEL), lambda t, te: (te[t], 0, 0)),
        ],
        out_specs=pl.BlockSpec((TILE, D_MODEL), lambda t, te: (t, 0)),
        scratch_shapes=[
            pltpu.VMEM((D_FF, D_MODEL), jnp.bfloat16),
            pltpu.VMEM((D_MODEL, D_FF), jnp.bfloat16),
            pltpu.VMEM((TILE, D_MODEL), jnp.bfloat16),
            pltpu.VMEM((TILE, D_FF), jnp.bfloat16),
        ],
    )
    return pl.pallas_call(
        _gg_body,
        grid_spec=grid_spec,
        out_shape=jax.ShapeDtypeStruct((NP_PAD, D_MODEL), jnp.float32),
    )(te, xs, W1, b1.reshape(E, 1, D_FF), W2, b2.reshape(E, 1, D_MODEL))


_CTOK = N_TOK // (NC * NS)
_CSUB = 32
_NCB = _CTOK // _CSUB


def _combine_kernel(x_hbm, ys_hbm, pos_hbm, g_hbm, out_hbm,
                    p0_v, p1_v, g0_v, g1_v, bx_v, y0_v, y1_v, sem):
    cid = lax.axis_index("c")
    sid = lax.axis_index("s")
    wid = sid * NC + cid
    tbase = wid * _CTOK

    for sb in range(_NCB):
        t0 = tbase + sb * _CSUB
        pltpu.sync_copy(pos_hbm.at[pl.ds(t0, _CSUB)], p0_v.at[0])
        pltpu.sync_copy(pos_hbm.at[pl.ds(N_TOK + t0, _CSUB)], p1_v.at[0])
        pltpu.sync_copy(g_hbm.at[0, pl.ds(t0, _CSUB)], g0_v)
        pltpu.sync_copy(g_hbm.at[1, pl.ds(t0, _CSUB)], g1_v)
        pltpu.sync_copy(x_hbm.at[pl.ds(t0, _CSUB)], bx_v)
        pltpu.async_copy(ys_hbm.at[p0_v.at[0]], y0_v, sem).wait()
        pltpu.async_copy(ys_hbm.at[p1_v.at[0]], y1_v, sem).wait()

        def tok_body(t, carry):
            g0 = plsc.load_gather(g0_v, [jnp.full((LANES,), t, jnp.int32)])
            g1 = plsc.load_gather(g1_v, [jnp.full((LANES,), t, jnp.int32)])
            for v in range(D_MODEL // LANES):
                sl = pl.ds(v * LANES, LANES)
                bx_v[t, sl] = (bx_v[t, sl] + g0 * y0_v[t, sl]
                               + g1 * y1_v[t, sl])
            return carry

        lax.fori_loop(0, _CSUB, tok_body, 0)
        pltpu.sync_copy(bx_v, out_hbm.at[pl.ds(t0, _CSUB)])


@jax.jit
def _combine(x2d, ys, pos, g):
    mesh = plsc.VectorSubcoreMesh(core_axis_name="c", subcore_axis_name="s", num_cores=NC, num_subcores=NS)
    f = pl.kernel(
        _combine_kernel,
        mesh=mesh,
        compiler_params=pltpu.CompilerParams(needs_layout_passes=False),
        out_type=jax.ShapeDtypeStruct((N_TOK, D_MODEL), jnp.float32),
        scratch_types=[
            pltpu.VMEM((1, _CSUB), jnp.int32),
            pltpu.VMEM((1, _CSUB), jnp.int32),
            pltpu.VMEM((_CSUB,), jnp.float32),
            pltpu.VMEM((_CSUB,), jnp.float32),
            pltpu.VMEM((_CSUB, D_MODEL), jnp.float32),
            pltpu.VMEM((_CSUB, D_MODEL), jnp.float32),
            pltpu.VMEM((_CSUB, D_MODEL), jnp.float32),
            pltpu.SemaphoreType.DMA,
        ],
    )
    return f(x2d, ys, pos, g)


def kernel(x, W_router, W1, b1, W2, b2):
    B, L, D = x.shape
    x2d = x.reshape(B * L, D)
    e_top, g_top = _router(x2d, W_router)
    xs, pos, te, _ = _dispatch(e_top.reshape(NPAIR), x2d)
    ys = _grouped_gemm(te, xs, W1, b1, W2, b2)
    out = _combine(x2d, ys, pos, g_top)
    return out.reshape(B, L, D)


# speedup vs baseline: 1.4095x; 1.0492x over previous
import functools

import jax
import jax.numpy as jnp
from jax import lax
from jax.experimental import pallas as pl
from jax.experimental.pallas import tpu as pltpu
from jax.experimental.pallas import tpu_sc as plsc

E = 16
D_MODEL = 768
D_FF = 3072
N_TOK = 4096
K_TOP = 2
NPAIR = N_TOK * K_TOP
TILE = 256
NP_PAD = NPAIR + E * TILE
NT = NP_PAD // TILE

NC = 2
NS = 16
LANES = 16


_RT = 512


def _router_body(x_ref, wr_ref, eo_ref, go_ref):
    logits = jnp.dot(x_ref[...], wr_ref[...].T, preferred_element_type=jnp.float32)
    probs = jax.nn.softmax(logits, axis=-1)
    n = probs.shape[0]
    ids = lax.broadcasted_iota(jnp.int32, (n, E), 1)
    m1 = jnp.max(probs, axis=-1, keepdims=True)
    a1 = jnp.min(jnp.where(probs == m1, ids, E), axis=-1, keepdims=True)
    masked = jnp.where(ids == a1, -jnp.inf, probs)
    m2 = jnp.max(masked, axis=-1, keepdims=True)
    a2 = jnp.min(jnp.where(masked == m2, ids, E), axis=-1, keepdims=True)
    eo_ref[...] = jnp.concatenate([a1.T, a2.T], axis=0)
    go_ref[...] = jnp.concatenate([m1.T, m2.T], axis=0)


@jax.jit
def _router(x2d, W_router):
    grid = (N_TOK // _RT,)
    return pl.pallas_call(
        _router_body,
        grid=grid,
        in_specs=[
            pl.BlockSpec((_RT, D_MODEL), lambda t: (t, 0)),
            pl.BlockSpec((E, D_MODEL), lambda t: (0, 0)),
        ],
        out_specs=[
            pl.BlockSpec((K_TOP, _RT), lambda t: (0, t)),
            pl.BlockSpec((K_TOP, _RT), lambda t: (0, t)),
        ],
        out_shape=[
            jax.ShapeDtypeStruct((K_TOP, N_TOK), jnp.int32),
            jax.ShapeDtypeStruct((K_TOP, N_TOK), jnp.float32),
        ],
    )(x2d, W_router)


_CHUNK = NPAIR // NS
_VR = _CHUNK // LANES
_ROWS = _CHUNK // NC
_RSUB = 64
_NRB = _ROWS // _RSUB


def _dispatch_kernel(epair_hbm, x_hbm, xs_hbm, pos_hbm, te_hbm, stage_hbm,
                     ids_v, histall_v, cnt_v, start_v, ebase_v,
                     pos_v, posb_v, tok_v, row_v, te_v, sem):
    cid = lax.axis_index("c")
    sid = lax.axis_index("s")
    base = sid * _CHUNK
    lane_iota = lax.iota(jnp.int32, LANES)
    zero16 = jnp.zeros((LANES,), jnp.int32)

    pltpu.sync_copy(epair_hbm.at[pl.ds(base, _CHUNK)], ids_v)

    d0, _ = plsc.scan_count(zero16)
    off = d0 - lane_iota

    cnt_v[...] = zero16

    def rank_body(j, carry):
        v = ids_v[pl.ds(j * LANES, LANES)]
        dup, last = plsc.scan_count(v)
        b = plsc.load_gather(cnt_v, [v])
        p = b + dup - off
        pos_v[pl.ds(j * LANES, LANES)] = p
        plsc.store_scatter(cnt_v, [v], p + 1, mask=last)
        tok_v[pl.ds(j * LANES, LANES)] = (base + j * LANES + lane_iota) & (N_TOK - 1)
        return carry

    lax.fori_loop(0, _VR, rank_body, 0)

    pltpu.sync_copy(cnt_v, stage_hbm.at[cid, sid])
    plsc.subcore_barrier()
    pltpu.sync_copy(stage_hbm.at[cid], histall_v)

    totals = zero16
    before = zero16
    for w in range(NS):
        hw = histall_v[w]
        totals = totals + hw
        before = before + jnp.where(w < sid, hw, zero16)

    padded = (totals + (TILE - 1)) & (-TILE)
    ebase = plsc.cumsum(padded) - padded
    ebase_v[...] = ebase
    start_v[...] = ebase + before

    def fin_body(j, carry):
        v = ids_v[pl.ds(j * LANES, LANES)]
        p = pos_v[pl.ds(j * LANES, LANES)]
        gp = plsc.load_gather(start_v, [v]) + p
        pos_v[pl.ds(j * LANES, LANES)] = gp
        return carry

    lax.fori_loop(0, _VR, fin_body, 0)

    for b in range(NC * _NRB):
        for jj in range(_RSUB // LANES):
            posb_v[b, pl.ds(jj * LANES, LANES)] = \
                pos_v[pl.ds(b * _RSUB + jj * LANES, LANES)]

    @pl.when(cid == 0)
    def _write_pos():
        pltpu.sync_copy(pos_v, pos_hbm.at[pl.ds(base, _CHUNK)])

    for rb in range(_NRB):
        b = cid * _NRB + rb
        pltpu.async_copy(
            x_hbm.at[tok_v.at[pl.ds(b * _RSUB, _RSUB)]], row_v, sem).wait()
        pltpu.async_copy(row_v, xs_hbm.at[posb_v.at[b]], sem).wait()

    @pl.when(jnp.logical_and(cid == 0, sid == 0))
    def _tiles():
        for j in range(NT // LANES):
            tstart = (lane_iota + j * LANES) * TILE
            acc = jnp.full((LANES,), -1, jnp.int32)
            for e in range(E):
                be = jnp.take_along_axis(
                    ebase, jnp.full((LANES,), e, jnp.int32), axis=0)
                acc = acc + jnp.where(tstart >= be, 1, 0)
            te_v[pl.ds(j * LANES, LANES)] = acc
        pltpu.sync_copy(te_v, te_hbm)


@jax.jit
def _dispatch(epair, x2d):
    mesh = plsc.VectorSubcoreMesh(core_axis_name="c", subcore_axis_name="s", num_cores=NC, num_subcores=NS)
    f = pl.kernel(
        _dispatch_kernel,
        mesh=mesh,
        compiler_params=pltpu.CompilerParams(needs_layout_passes=False),
        out_type=[
            jax.ShapeDtypeStruct((NP_PAD, D_MODEL), jnp.float32),
            jax.ShapeDtypeStruct((NPAIR,), jnp.int32),
            jax.ShapeDtypeStruct((NT,), jnp.int32),
            jax.ShapeDtypeStruct((NC, NS, LANES), jnp.int32),
        ],
        scratch_types=[
            pltpu.VMEM((_CHUNK,), jnp.int32),
            pltpu.VMEM((NS, LANES), jnp.int32),
            pltpu.VMEM((LANES,), jnp.int32),
            pltpu.VMEM((LANES,), jnp.int32),
            pltpu.VMEM((LANES,), jnp.int32),
            pltpu.VMEM((_CHUNK,), jnp.int32),
            pltpu.VMEM((_NRB * NC, _RSUB), jnp.int32),
            pltpu.VMEM((_CHUNK,), jnp.int32),
            pltpu.VMEM((_RSUB, D_MODEL), jnp.float32),
            pltpu.VMEM((NT,), jnp.int32),
            pltpu.SemaphoreType.DMA,
        ],
    )
    return f(epair, x2d)


def _gg_body(te_ref, xs_ref, w1_ref, b1_ref, w2_ref, b2_ref, ys_ref):
    h = jnp.dot(xs_ref[...], w1_ref[0].T, preferred_element_type=jnp.float32,
                precision=lax.Precision.DEFAULT)
    h = h + b1_ref[0]
    h = 0.5 * h * (1.0 + lax.erf(h * 0.7071067811865476))
    o = jnp.dot(h, w2_ref[0].T, preferred_element_type=jnp.float32,
                precision=lax.Precision.DEFAULT)
    ys_ref[...] = o + b2_ref[0]


@jax.jit
def _grouped_gemm(te, xs, W1, b1, W2, b2):
    grid_spec = pltpu.PrefetchScalarGridSpec(
        num_scalar_prefetch=1,
        grid=(NT,),
        in_specs=[
            pl.BlockSpec((TILE, D_MODEL), lambda t, te: (t, 0)),
            pl.BlockSpec((1, D_FF, D_MODEL), lambda t, te: (te[t], 0, 0)),
            pl.BlockSpec((1, 1, D_FF), lambda t, te: (te[t], 0, 0)),
            pl.BlockSpec((1, D_MODEL, D_FF), lambda t, te: (te[t], 0, 0)),
            pl.BlockSpec((1, 1, D_MODEL), lambda t, te: (te[t], 0, 0)),
        ],
        out_specs=pl.BlockSpec((TILE, D_MODEL), lambda t, te: (t, 0)),
    )
    return pl.pallas_call(
        _gg_body,
        grid_spec=grid_spec,
        out_shape=jax.ShapeDtypeStruct((NP_PAD, D_MODEL), jnp.float32),
    )(te, xs, W1, b1.reshape(E, 1, D_FF), W2, b2.reshape(E, 1, D_MODEL))


_CTOK = N_TOK // (NC * NS)
_CSUB = 32
_NCB = _CTOK // _CSUB


def _combine_kernel(x_hbm, ys_hbm, pos_hbm, g_hbm, out_hbm,
                    p0_v, p1_v, g0_v, g1_v, bx_v, y0_v, y1_v, sem):
    cid = lax.axis_index("c")
    sid = lax.axis_index("s")
    wid = sid * NC + cid
    tbase = wid * _CTOK

    for sb in range(_NCB):
        t0 = tbase + sb * _CSUB
        pltpu.sync_copy(pos_hbm.at[pl.ds(t0, _CSUB)], p0_v.at[0])
        pltpu.sync_copy(pos_hbm.at[pl.ds(N_TOK + t0, _CSUB)], p1_v.at[0])
        pltpu.sync_copy(g_hbm.at[0, pl.ds(t0, _CSUB)], g0_v)
        pltpu.sync_copy(g_hbm.at[1, pl.ds(t0, _CSUB)], g1_v)
        pltpu.sync_copy(x_hbm.at[pl.ds(t0, _CSUB)], bx_v)
        pltpu.async_copy(ys_hbm.at[p0_v.at[0]], y0_v, sem).wait()
        pltpu.async_copy(ys_hbm.at[p1_v.at[0]], y1_v, sem).wait()

        def tok_body(t, carry):
            g0 = plsc.load_gather(g0_v, [jnp.full((LANES,), t, jnp.int32)])
            g1 = plsc.load_gather(g1_v, [jnp.full((LANES,), t, jnp.int32)])
            for v in range(D_MODEL // LANES):
                sl = pl.ds(v * LANES, LANES)
                bx_v[t, sl] = (bx_v[t, sl] + g0 * y0_v[t, sl]
                               + g1 * y1_v[t, sl])
            return carry

        lax.fori_loop(0, _CSUB, tok_body, 0)
        pltpu.sync_copy(bx_v, out_hbm.at[pl.ds(t0, _CSUB)])


@jax.jit
def _combine(x2d, ys, pos, g):
    mesh = plsc.VectorSubcoreMesh(core_axis_name="c", subcore_axis_name="s", num_cores=NC, num_subcores=NS)
    f = pl.kernel(
        _combine_kernel,
        mesh=mesh,
        compiler_params=pltpu.CompilerParams(needs_layout_passes=False),
        out_type=jax.ShapeDtypeStruct((N_TOK, D_MODEL), jnp.float32),
        scratch_types=[
            pltpu.VMEM((1, _CSUB), jnp.int32),
            pltpu.VMEM((1, _CSUB), jnp.int32),
            pltpu.VMEM((_CSUB,), jnp.float32),
            pltpu.VMEM((_CSUB,), jnp.float32),
            pltpu.VMEM((_CSUB, D_MODEL), jnp.float32),
            pltpu.VMEM((_CSUB, D_MODEL), jnp.float32),
            pltpu.VMEM((_CSUB, D_MODEL), jnp.float32),
            pltpu.SemaphoreType.DMA,
        ],
    )
    return f(x2d, ys, pos, g)


def kernel(x, W_router, W1, b1, W2, b2):
    B, L, D = x.shape
    x2d = x.reshape(B * L, D)
    e_top, g_top = _router(x2d, W_router)
    xs, pos, te, _ = _dispatch(e_top.reshape(NPAIR), x2d)
    ys = _grouped_gemm(te, xs, W1, b1, W2, b2)
    out = _combine(x2d, ys, pos, g_top)
    return out.reshape(B, L, D)
